# baseline (speedup 1.0000x reference)
import functools
import jax
import jax.numpy as jnp
from jax import lax
from jax.experimental import pallas as pl
from jax.experimental.pallas import tpu as pltpu
from jax.experimental.pallas import tpu_sc as plsc

_N = 10000
_F = 128
_NC = 2
_NS = 16
_NW = _NC * _NS
_CHUNK = 128
_NACC = 10240
_ZROWS = _NACC // _NS
_WB = [(k * _CHUNK, _CHUNK) for k in range(_ZROWS // _CHUNK)]


def _seg_sum(h, src_t, dst_t, half):
    mesh = plsc.VectorSubcoreMesh(core_axis_name="c", subcore_axis_name="s")

    @functools.partial(
        pl.kernel,
        out_type=jax.ShapeDtypeStruct((_NC, _NACC, _F), jnp.float32),
        mesh=mesh,
        scratch_types=[
            pltpu.VMEM((half, _CHUNK), jnp.int32),
            pltpu.VMEM((half, _CHUNK), jnp.int32),
            pltpu.VMEM((_CHUNK, _F), jnp.float32),
            pltpu.VMEM((_CHUNK, _F), jnp.float32),
            pltpu.VMEM_SHARED((_NACC, _F), jnp.float32),
            pltpu.SemaphoreType.DMA,
            pltpu.SemaphoreType.DMA,
        ],
    )
    def seg(h_hbm, src_hbm, dst_hbm, out_hbm, src_v, dst_v, rows0, rows1, acc,
            sem0, sem1):
        c = lax.axis_index("c")
        s = lax.axis_index("s")
        wid = s * _NC + c

        @pl.loop(0, _CHUNK)
        def _(i):
            @pl.loop(0, _F, step=16)
            def _(j):
                rows0[i, pl.ds(j, 16)] = jnp.zeros((16,), jnp.float32)

        zbase = s * _ZROWS
        for off, sz in _WB:
            pltpu.sync_copy(rows0.at[pl.ds(0, sz)],
                            acc.at[pl.ds(zbase + off, sz)])

        plsc.subcore_barrier()

        @pl.when(c == 1)
        def _():
            for stage in range(4):
                pltpu.sync_copy(src_hbm.at[s].at[stage], src_v)
                pltpu.sync_copy(dst_hbm.at[s].at[stage], dst_v)

                pltpu.async_copy(h_hbm.at[src_v.at[0]], rows0, sem0)

                @pl.loop(0, half, step=2)
                def _(j):
                    pltpu.async_copy(h_hbm.at[src_v.at[j + 1]], rows1, sem1)
                    pltpu.make_async_copy(h_hbm.at[src_v.at[j]], rows0, sem0).wait()
                    pltpu.sync_copy(rows0, acc.at[dst_v.at[j]], add=True)

                    @pl.when(j + 2 < half)
                    def _():
                        pltpu.async_copy(h_hbm.at[src_v.at[j + 2]], rows0, sem0)

                    pltpu.make_async_copy(h_hbm.at[src_v.at[j + 1]], rows1, sem1).wait()
                    pltpu.sync_copy(rows1, acc.at[dst_v.at[j + 1]], add=True)

        plsc.subcore_barrier()

        for off, sz in _WB:
            ob = s * _ZROWS + off
            pltpu.sync_copy(acc.at[pl.ds(ob, sz)], rows0.at[pl.ds(0, sz)])
            pltpu.sync_copy(rows0.at[pl.ds(0, sz)], out_hbm.at[c].at[pl.ds(ob, sz)])

    return seg(h, src_t, dst_t)


def _mm_relu(p, w):
    BN = 2048

    def body(p_ref, w_ref, o_ref):
        x = p_ref[0] + p_ref[1]
        o_ref[...] = jnp.maximum(
            jnp.dot(x, w_ref[...], preferred_element_type=jnp.float32), 0.0)

    return pl.pallas_call(
        body,
        grid=(_NACC // BN,),
        in_specs=[
            pl.BlockSpec((_NC, BN, _F), lambda i: (0, i, 0)),
            pl.BlockSpec((_F, _F), lambda i: (0, 0)),
        ],
        out_specs=pl.BlockSpec((BN, _F), lambda i: (i, 0)),
        out_shape=jax.ShapeDtypeStruct((_NACC, _F), jnp.float32),
    )(p, w)


def _mm_head(p, w):
    BN = 2048

    def body(p_ref, w_ref, lp_ref, h_ref):
        x = p_ref[0] + p_ref[1]
        h = jnp.dot(x, w_ref[...], preferred_element_type=jnp.float32)
        h_ref[...] = h
        m = jnp.max(h, axis=1, keepdims=True)
        lse = jnp.log(jnp.sum(jnp.exp(h - m), axis=1, keepdims=True)) + m
        lp_ref[...] = h - lse

    return pl.pallas_call(
        body,
        grid=(_NACC // BN,),
        in_specs=[
            pl.BlockSpec((_NC, BN, _F), lambda i: (0, i, 0)),
            pl.BlockSpec((_F, _F), lambda i: (0, 0)),
        ],
        out_specs=[
            pl.BlockSpec((BN, _F), lambda i: (i, 0)),
            pl.BlockSpec((BN, _F), lambda i: (i, 0)),
        ],
        out_shape=[
            jax.ShapeDtypeStruct((_NACC, _F), jnp.float32),
            jax.ShapeDtypeStruct((_NACC, _F), jnp.float32),
        ],
    )(p, w)


def kernel(tinput, adj, W0, W1, W2):
    E = adj.shape[1]
    epb = 8 * _NS * _CHUNK
    E_pad = ((E + epb - 1) // epb) * epb
    n_chunks = E_pad // (_NS * _CHUNK)
    half = n_chunks // 4
    pad = E_pad - E
    src = jnp.concatenate([adj[0], jnp.zeros((pad,), jnp.int32)])
    dst = jnp.concatenate([adj[1], jnp.full((pad,), _N, jnp.int32)])
    src_t = src.reshape(_NS, 4, half, _CHUNK)
    dst_t = dst.reshape(_NS, 4, half, _CHUNK)

    h = tinput
    for w in (W0, W1):
        p = _seg_sum(h, src_t, dst_t, half)
        h = _mm_relu(p, w)
    p = _seg_sum(h, src_t, dst_t, half)
    lp, h3 = _mm_head(p, W2)
    return (lp[:_N], h3[:_N])


# speedup vs baseline: 1.1998x; 1.1998x over previous
import functools
import jax
import jax.numpy as jnp
from jax import lax
from jax.experimental import pallas as pl
from jax.experimental.pallas import tpu as pltpu
from jax.experimental.pallas import tpu_sc as plsc

_N = 10000
_F = 128
_NC = 2
_NS = 16
_NW = _NC * _NS
_CHUNK = 128
_NACC = 10240
_ZROWS = _NACC // _NS
_WB = [(k * _CHUNK, _CHUNK) for k in range(_ZROWS // _CHUNK)]


_STAGE = 40


def _stages(q):
    out, off = [], 0
    while off < q:
        sz = min(_STAGE, q - off)
        out.append((off, sz))
        off += sz
    assert all(sz % 2 == 0 and off % 8 == 0 for off, sz in out)
    return out


def _seg_sum(h, src_t, dst_t, q0, q1):
    mesh = plsc.VectorSubcoreMesh(core_axis_name="c", subcore_axis_name="s")

    @functools.partial(
        pl.kernel,
        out_type=jax.ShapeDtypeStruct((_NC, _NACC, _F), jnp.float32),
        mesh=mesh,
        scratch_types=[
            pltpu.VMEM((_STAGE, _CHUNK), jnp.int32),
            pltpu.VMEM((_STAGE, _CHUNK), jnp.int32),
            pltpu.VMEM((_CHUNK, _F), jnp.float32),
            pltpu.VMEM((_CHUNK, _F), jnp.float32),
            pltpu.VMEM_SHARED((_NACC, _F), jnp.float32),
            pltpu.SemaphoreType.DMA,
            pltpu.SemaphoreType.DMA,
        ],
    )
    def seg(h_hbm, src_hbm, dst_hbm, out_hbm, src_v, dst_v, rows0, rows1, acc,
            sem0, sem1):
        c = lax.axis_index("c")
        s = lax.axis_index("s")

        @pl.loop(0, _CHUNK)
        def _(i):
            @pl.loop(0, _F, step=16)
            def _(j):
                rows0[i, pl.ds(j, 16)] = jnp.zeros((16,), jnp.float32)

        zbase = s * _ZROWS
        for off, sz in _WB:
            pltpu.sync_copy(rows0.at[pl.ds(0, sz)],
                            acc.at[pl.ds(zbase + off, sz)])

        plsc.subcore_barrier()

        def run(my_base, my_q):
            def emit(off, sz):
                base = my_base + s * my_q + off
                pltpu.sync_copy(src_hbm.at[pl.ds(base, sz)], src_v.at[pl.ds(0, sz)])
                pltpu.sync_copy(dst_hbm.at[pl.ds(base, sz)], dst_v.at[pl.ds(0, sz)])

                pltpu.async_copy(h_hbm.at[src_v.at[0]], rows0, sem0)

                @pl.loop(0, sz, step=2)
                def _(j):
                    pltpu.async_copy(h_hbm.at[src_v.at[j + 1]], rows1, sem1)
                    pltpu.make_async_copy(h_hbm.at[src_v.at[j]], rows0, sem0).wait()
                    pltpu.sync_copy(rows0, acc.at[dst_v.at[j]], add=True)

                    @pl.when(j + 2 < sz)
                    def _():
                        pltpu.async_copy(h_hbm.at[src_v.at[j + 2]], rows0, sem0)

                    pltpu.make_async_copy(h_hbm.at[src_v.at[j + 1]], rows1, sem1).wait()
                    pltpu.sync_copy(rows1, acc.at[dst_v.at[j + 1]], add=True)

            for off, sz in _stages(my_q):
                emit(off, sz)

        @pl.when(c == 0)
        def _():
            run(0, q0)

        @pl.when(c == 1)
        def _():
            run(_NS * q0, q1)

        plsc.subcore_barrier()

        for off, sz in _WB:
            ob = s * _ZROWS + off
            pltpu.sync_copy(acc.at[pl.ds(ob, sz)], rows0.at[pl.ds(0, sz)])
            pltpu.sync_copy(rows0.at[pl.ds(0, sz)], out_hbm.at[c].at[pl.ds(ob, sz)])

    return seg(h, src_t, dst_t)


_Q0 = 128
_Q1 = 32


def _mm_relu(p, w):
    BN = 2048

    def body(p_ref, w_ref, o_ref):
        x = p_ref[0] + p_ref[1]
        o_ref[...] = jnp.maximum(
            jnp.dot(x, w_ref[...], preferred_element_type=jnp.float32), 0.0)

    return pl.pallas_call(
        body,
        grid=(_NACC // BN,),
        in_specs=[
            pl.BlockSpec((_NC, BN, _F), lambda i: (0, i, 0)),
            pl.BlockSpec((_F, _F), lambda i: (0, 0)),
        ],
        out_specs=pl.BlockSpec((BN, _F), lambda i: (i, 0)),
        out_shape=jax.ShapeDtypeStruct((_NACC, _F), jnp.float32),
    )(p, w)


def _mm_head(p, w):
    BN = 2048

    def body(p_ref, w_ref, lp_ref, h_ref):
        x = p_ref[0] + p_ref[1]
        h = jnp.dot(x, w_ref[...], preferred_element_type=jnp.float32)
        h_ref[...] = h
        m = jnp.max(h, axis=1, keepdims=True)
        lse = jnp.log(jnp.sum(jnp.exp(h - m), axis=1, keepdims=True)) + m
        lp_ref[...] = h - lse

    return pl.pallas_call(
        body,
        grid=(_NACC // BN,),
        in_specs=[
            pl.BlockSpec((_NC, BN, _F), lambda i: (0, i, 0)),
            pl.BlockSpec((_F, _F), lambda i: (0, 0)),
        ],
        out_specs=[
            pl.BlockSpec((BN, _F), lambda i: (i, 0)),
            pl.BlockSpec((BN, _F), lambda i: (i, 0)),
        ],
        out_shape=[
            jax.ShapeDtypeStruct((_NACC, _F), jnp.float32),
            jax.ShapeDtypeStruct((_NACC, _F), jnp.float32),
        ],
    )(p, w)


def kernel(tinput, adj, W0, W1, W2):
    E = adj.shape[1]
    T = _NS * (_Q0 + _Q1)
    E_pad = T * _CHUNK
    assert E_pad >= E
    pad = E_pad - E
    src = jnp.concatenate([adj[0], jnp.zeros((pad,), jnp.int32)])
    dst = jnp.concatenate([adj[1], jnp.full((pad,), _N, jnp.int32)])
    src_t = src.reshape(T, _CHUNK)
    dst_t = dst.reshape(T, _CHUNK)

    h = tinput
    for w in (W0, W1):
        p = _seg_sum(h, src_t, dst_t, _Q0, _Q1)
        h = _mm_relu(p, w)
    p = _seg_sum(h, src_t, dst_t, _Q0, _Q1)
    lp, h3 = _mm_head(p, W2)
    return (lp[:_N], h3[:_N])
